# Initial kernel scaffold; baseline (speedup 1.0000x reference)
import jax
import jax.numpy as jnp
from jax.experimental import pallas as pl


def kernel(input, graph):
    raise NotImplementedError("write your pallas kernel here")


# speedup vs baseline: 68.8132x; 68.8132x over previous
import functools

import jax
import jax.numpy as jnp
from jax import lax
from jax.experimental import pallas as pl
from jax.experimental.pallas import tpu as pltpu
from jax.experimental.pallas import tpu_sc as plsc

N = 32768
D = 512
S = 256

L = 16
G = D // L
C = 32
NW = 32
SEG_PER_W = S // NW


def _sc_body(x_hbm, graph_hbm, out_hbm, graph_v, buf_v, out_v):
    cid = lax.axis_index("c")
    sid = lax.axis_index("s")
    wid = sid * 2 + cid

    pltpu.sync_copy(graph_hbm, graph_v)
    base_seg = wid * SEG_PER_W

    for si in range(SEG_PER_W):
        seg = base_seg + si
        se = graph_v[pl.ds(seg * 2, L)]
        s = se[0]
        e = se[1]
        count = e - s + 1
        astart = s - lax.rem(s, 8)
        nch = lax.div(e + 1 - astart + (C - 1), C)

        def chunk_body(k, acc, s=s, e=e, astart=astart):
            start = astart + k * C
            start_c = pl.multiple_of(jnp.minimum(start, N - C), 8)
            pltpu.sync_copy(x_hbm.at[pl.ds(start_c, C)], buf_v)
            lo = jnp.maximum(s, start) - start_c
            hi = jnp.minimum(e + 1, start + C) - start_c

            def row_body(r, a):
                return tuple(a[g] + buf_v[r, pl.ds(g * L, L)] for g in range(G))

            return lax.fori_loop(lo, hi, row_body, acc)

        acc0 = tuple(jnp.zeros((L,), jnp.float32) for _ in range(G))
        acc = lax.fori_loop(0, nch, chunk_body, acc0)

        cnt_v = jnp.full((L,), count, jnp.int32).astype(jnp.float32)
        inv = jnp.full((L,), 1.0, jnp.float32) / cnt_v
        for g in range(G):
            out_v[si, pl.ds(g * L, L)] = acc[g] * inv + 0.006

    pltpu.sync_copy(out_v, out_hbm.at[pl.ds(base_seg, SEG_PER_W)])


@jax.jit
def _wise_pooling(x, graph):
    mesh = plsc.VectorSubcoreMesh(core_axis_name="c", subcore_axis_name="s")
    f = pl.kernel(
        _sc_body,
        out_type=jax.ShapeDtypeStruct((S, D), jnp.float32),
        mesh=mesh,
        scratch_types=[
            pltpu.VMEM((S * 2 + L,), jnp.int32),
            pltpu.VMEM((C, D), jnp.float32),
            pltpu.VMEM((SEG_PER_W, D), jnp.float32),
        ],
    )
    return f(x, graph)


def kernel(input, graph):
    gflat = jnp.pad(graph.astype(jnp.int32).reshape(-1), (0, L))
    return _wise_pooling(input, gflat)
